# Initial kernel scaffold; baseline (speedup 1.0000x reference)
import jax
import jax.numpy as jnp
from jax.experimental import pallas as pl


def kernel(x, centroids):
    raise NotImplementedError("write your pallas kernel here")


# speedup vs baseline: 1.9979x; 1.9979x over previous
import functools

import jax
import jax.numpy as jnp
from jax import lax
from jax.experimental import pallas as pl
from jax.experimental.pallas import tpu as pltpu
from jax.experimental.pallas import tpu_sc as plsc

N = 64 * 1024
C = 32
K = 512
BN = 2048

NUM_CORES = 2
NUM_SUBCORES = 16
NW = NUM_CORES * NUM_SUBCORES
B_PER_W = N // NW
IDX_CHUNK = 128
N_CHUNKS = B_PER_W // IDX_CHUNK


def _argmin_body(x_ref, ct_ref, b2_ref, idx_ref):
    xb = x_ref[...]
    ct = ct_ref[...]
    b2 = b2_ref[...]
    xc = jnp.dot(xb, ct, preferred_element_type=jnp.float32)
    a2 = jnp.sum(xb * xb, axis=1, keepdims=True)
    d2 = (a2 + b2) - 2.0 * xc
    d2 = jnp.maximum(d2, 0.0)
    m = jnp.min(d2, axis=1, keepdims=True)
    kio = lax.broadcasted_iota(jnp.int32, d2.shape, 1)
    idx = jnp.min(jnp.where(d2 <= m, kio, K), axis=1, keepdims=True)
    idx_ref[...] = idx


def _tc_argmin(xf, cT, b2):
    return pl.pallas_call(
        _argmin_body,
        grid=(N // BN,),
        in_specs=[
            pl.BlockSpec((BN, C), lambda i: (i, 0)),
            pl.BlockSpec((C, K), lambda i: (0, 0)),
            pl.BlockSpec((1, K), lambda i: (0, 0)),
        ],
        out_specs=pl.BlockSpec((BN, 1), lambda i: (i, 0)),
        out_shape=jax.ShapeDtypeStruct((N, 1), jnp.int32),
        compiler_params=pltpu.CompilerParams(
            dimension_semantics=("arbitrary",),
        ),
    )(xf, cT, b2)


def _sc_gather_body(table_hbm, idx_hbm, out_hbm, idx_v, rows_v, sem):
    wid = lax.axis_index("s") * NUM_CORES + lax.axis_index("c")
    base = wid * B_PER_W
    pltpu.sync_copy(idx_hbm.at[pl.ds(wid * N_CHUNKS, N_CHUNKS)], idx_v)
    copies = []
    for j in range(N_CHUNKS):
        copies.append(
            pltpu.async_copy(
                table_hbm.at[idx_v.at[j]],
                rows_v.at[pl.ds(j * IDX_CHUNK, IDX_CHUNK)],
                sem,
            )
        )
    for cp in copies:
        cp.wait()
    pltpu.sync_copy(rows_v, out_hbm.at[pl.ds(base, B_PER_W)])


@functools.cache
def _sc_gather():
    return pl.kernel(
        _sc_gather_body,
        out_type=jax.ShapeDtypeStruct((N, C), jnp.float32),
        mesh=plsc.VectorSubcoreMesh(core_axis_name="c", subcore_axis_name="s"),
        scratch_types=[
            pltpu.VMEM((N_CHUNKS, IDX_CHUNK), jnp.int32),
            pltpu.VMEM((B_PER_W, C), jnp.float32),
            pltpu.SemaphoreType.DMA,
        ],
        compiler_params=pltpu.CompilerParams(use_tc_tiling_on_sc=False),
    )


def kernel(x, centroids):
    xf = x.reshape(N, C)
    cT = centroids.T
    b2 = jnp.sum(centroids * centroids, axis=1)[None, :]
    idx = _tc_argmin(xf, cT, b2)
    idx2d = idx.reshape(NW * N_CHUNKS, IDX_CHUNK)
    out = _sc_gather()(centroids, idx2d)
    return out.reshape(x.shape)
